# baseline (device time: 14191 ns/iter reference)
import jax
import jax.numpy as jnp
from jax import lax
from jax.experimental import pallas as pl
from jax.experimental.pallas import tpu as pltpu

N_DEV = 4
B, SQ, SKV, DH, D_MODEL = 2, 128, 128, 64, 512
H_LOC = 4
QR = 64


def kernel(x, Wq, K_ext, V_ext, Wo):
    my = lax.axis_index("i")
    K_loc = lax.dynamic_slice_in_dim(K_ext, my * H_LOC, H_LOC, axis=2)
    V_loc = lax.dynamic_slice_in_dim(V_ext, my * H_LOC, H_LOC, axis=2)
    K_loc = jnp.transpose(K_loc, (0, 2, 1, 3)).astype(jnp.bfloat16)
    V_loc = jnp.transpose(V_loc, (0, 2, 1, 3)).astype(jnp.bfloat16)
    x2 = x.reshape(B * SQ, D_MODEL).astype(jnp.bfloat16)
    Wq16 = Wq.astype(jnp.bfloat16)
    Wo16 = Wo.astype(jnp.bfloat16)

    def body(x_ref, wq_ref, k_ref, v_ref, wo_ref, out_ref,
             comm_ref, fsum_ref, send_sems, recv_sems, out_sems):
        my_pos = lax.axis_index("i")
        xp = my_pos ^ 1
        yp = 3 - my_pos

        barrier_sem = pltpu.get_barrier_semaphore()
        for nbr in (xp, yp):
            pl.semaphore_signal(
                barrier_sem, inc=1,
                device_id=(nbr,), device_id_type=pl.DeviceIdType.MESH,
            )

        def exchange(src_slot, dst_slot, sem_idx, partner):
            return pltpu.make_async_remote_copy(
                src_ref=comm_ref.at[src_slot],
                dst_ref=comm_ref.at[dst_slot],
                send_sem=send_sems.at[sem_idx],
                recv_sem=recv_sems.at[sem_idx],
                device_id=(partner,),
                device_id_type=pl.DeviceIdType.MESH,
            )

        q2 = (jnp.dot(x_ref[:], wq_ref[:],
                      preferred_element_type=jnp.float32)
              * 0.125).astype(jnp.bfloat16)

        stage1 = {}
        stage2 = {}
        for b in range(B):
            ss = []
            for h in range(H_LOC):
                qbh = q2[b * SQ:(b + 1) * SQ, h * DH:(h + 1) * DH]
                ss.append(lax.dot_general(
                    qbh, k_ref[b, h], (((1,), (1,)), ((), ())),
                    preferred_element_type=jnp.float32,
                ))
            s_all = jnp.concatenate(ss, axis=0)
            m = jnp.max(s_all, axis=-1, keepdims=True)
            e = jnp.exp(s_all - m)
            w_all = (e / jnp.sum(e, axis=-1, keepdims=True)
                     ).astype(jnp.bfloat16)
            ctxs = [
                jnp.dot(w_all[h * SQ:(h + 1) * SQ, :], v_ref[b, h],
                        preferred_element_type=jnp.float32)
                for h in range(H_LOC)
            ]
            ctx_b = jnp.concatenate(ctxs, axis=-1).astype(jnp.bfloat16)
            p_b = jnp.dot(ctx_b, wo_ref[:],
                          preferred_element_type=jnp.float32)
            if b == 0:
                pl.semaphore_wait(barrier_sem, 2)
            s1_partner = xp if b == 0 else yp
            for j in range(2):
                qi = 2 * b + j
                comm_ref[4 * qi] = p_b[j * QR:(j + 1) * QR, :].astype(
                    jnp.bfloat16)
                r = exchange(4 * qi, 4 * qi + 1, 2 * qi, s1_partner)
                r.start()
                stage1[qi] = r

        for qi in range(4):
            stage1[qi].wait_recv()
            comm_ref[4 * qi + 2] = (
                comm_ref[4 * qi].astype(jnp.float32)
                + comm_ref[4 * qi + 1].astype(jnp.float32)
            ).astype(jnp.bfloat16)
            r = exchange(4 * qi + 2, 4 * qi + 3, 2 * qi + 1,
                         yp if qi < 2 else xp)
            r.start()
            stage2[qi] = r

        out_copies = []
        for qi in range(4):
            stage2[qi].wait_recv()
            b, j = divmod(qi, 2)
            fsum_ref[qi] = (
                comm_ref[4 * qi + 2].astype(jnp.float32)
                + comm_ref[4 * qi + 3].astype(jnp.float32)
            )
            c = pltpu.make_async_copy(
                fsum_ref.at[qi],
                out_ref.at[b, pl.ds(j * QR, QR), :],
                out_sems.at[qi],
            )
            c.start()
            out_copies.append(c)

        for c in out_copies:
            c.wait()
        for r in list(stage1.values()) + list(stage2.values()):
            r.wait_send()

    return pl.pallas_call(
        body,
        out_shape=jax.ShapeDtypeStruct((B, SQ, D_MODEL), jnp.float32),
        in_specs=[pl.BlockSpec(memory_space=pltpu.VMEM)] * 5,
        out_specs=pl.BlockSpec(memory_space=pltpu.MemorySpace.HBM),
        scratch_shapes=[
            pltpu.VMEM((16, QR, D_MODEL), jnp.bfloat16),
            pltpu.VMEM((4, QR, D_MODEL), jnp.float32),
            pltpu.SemaphoreType.DMA((8,)),
            pltpu.SemaphoreType.DMA((8,)),
            pltpu.SemaphoreType.DMA((4,)),
        ],
        compiler_params=pltpu.CompilerParams(collective_id=0),
    )(x2, Wq16, K_loc, V_loc, Wo16)


# device time: 12763 ns/iter; 1.1119x vs baseline; 1.1119x over previous
import jax
import jax.numpy as jnp
from jax import lax
from jax.experimental import pallas as pl
from jax.experimental.pallas import tpu as pltpu

N_DEV = 4
B, SQ, SKV, DH, D_MODEL = 2, 128, 128, 64, 512
H_LOC = 4
QR = 64


def kernel(x, Wq, K_ext, V_ext, Wo):
    my = lax.axis_index("i")
    K_loc = lax.dynamic_slice_in_dim(K_ext, my * H_LOC, H_LOC, axis=2)
    V_loc = lax.dynamic_slice_in_dim(V_ext, my * H_LOC, H_LOC, axis=2)
    K_loc = jnp.transpose(K_loc, (0, 2, 1, 3))
    V_loc = jnp.transpose(V_loc, (0, 2, 1, 3))
    x2 = x.reshape(B * SQ, D_MODEL)

    def body(x_ref, wq_ref, k_ref, v_ref, wo_ref, out_ref,
             comm_ref, fsum_ref, send_sems, recv_sems, out_sems):
        my_pos = lax.axis_index("i")
        xp = my_pos ^ 1
        yp = 3 - my_pos

        barrier_sem = pltpu.get_barrier_semaphore()
        for nbr in (xp, yp):
            pl.semaphore_signal(
                barrier_sem, inc=1,
                device_id=(nbr,), device_id_type=pl.DeviceIdType.MESH,
            )

        def exchange(src_slot, dst_slot, sem_idx, partner):
            return pltpu.make_async_remote_copy(
                src_ref=comm_ref.at[src_slot],
                dst_ref=comm_ref.at[dst_slot],
                send_sem=send_sems.at[sem_idx],
                recv_sem=recv_sems.at[sem_idx],
                device_id=(partner,),
                device_id_type=pl.DeviceIdType.MESH,
            )

        q2 = jnp.dot(x_ref[:], wq_ref[:],
                     preferred_element_type=jnp.float32) * 0.125

        stage1 = {}
        stage2 = {}
        for b in range(B):
            ss = []
            for h in range(H_LOC):
                qbh = q2[b * SQ:(b + 1) * SQ, h * DH:(h + 1) * DH]
                ss.append(lax.dot_general(
                    qbh, k_ref[b, h], (((1,), (1,)), ((), ())),
                    preferred_element_type=jnp.float32,
                ))
            s_all = jnp.concatenate(ss, axis=0)
            m = jnp.max(s_all, axis=-1, keepdims=True)
            e = jnp.exp(s_all - m)
            w_all = e / jnp.sum(e, axis=-1, keepdims=True)
            ctxs = [
                jnp.dot(w_all[h * SQ:(h + 1) * SQ, :], v_ref[b, h],
                        preferred_element_type=jnp.float32)
                for h in range(H_LOC)
            ]
            ctx_b = jnp.concatenate(ctxs, axis=-1)
            p_b = jnp.dot(ctx_b, wo_ref[:],
                          preferred_element_type=jnp.float32)
            if b == 0:
                pl.semaphore_wait(barrier_sem, 2)
            for j in range(2):
                qi = 2 * b + j
                comm_ref[4 * qi] = p_b[j * QR:(j + 1) * QR, :].astype(
                    jnp.bfloat16)
                r = exchange(4 * qi, 4 * qi + 1, 2 * qi,
                             xp if qi % 2 == 0 else yp)
                r.start()
                stage1[qi] = r

        for qi in range(4):
            stage1[qi].wait_recv()
            comm_ref[4 * qi + 2] = (
                comm_ref[4 * qi].astype(jnp.float32)
                + comm_ref[4 * qi + 1].astype(jnp.float32)
            ).astype(jnp.bfloat16)
            r = exchange(4 * qi + 2, 4 * qi + 3, 2 * qi + 1,
                         yp if qi % 2 == 0 else xp)
            r.start()
            stage2[qi] = r

        out_copies = []
        for qi in range(4):
            stage2[qi].wait_recv()
            b, j = divmod(qi, 2)
            fsum_ref[qi] = (
                comm_ref[4 * qi + 2].astype(jnp.float32)
                + comm_ref[4 * qi + 3].astype(jnp.float32)
            )
            c = pltpu.make_async_copy(
                fsum_ref.at[qi],
                out_ref.at[b, pl.ds(j * QR, QR), :],
                out_sems.at[qi],
            )
            c.start()
            out_copies.append(c)

        for c in out_copies:
            c.wait()
        for r in list(stage1.values()) + list(stage2.values()):
            r.wait_send()

    return pl.pallas_call(
        body,
        out_shape=jax.ShapeDtypeStruct((B, SQ, D_MODEL), jnp.float32),
        in_specs=[pl.BlockSpec(memory_space=pltpu.VMEM)] * 5,
        out_specs=pl.BlockSpec(memory_space=pltpu.MemorySpace.HBM),
        scratch_shapes=[
            pltpu.VMEM((16, QR, D_MODEL), jnp.bfloat16),
            pltpu.VMEM((4, QR, D_MODEL), jnp.float32),
            pltpu.SemaphoreType.DMA((8,)),
            pltpu.SemaphoreType.DMA((8,)),
            pltpu.SemaphoreType.DMA((4,)),
        ],
        compiler_params=pltpu.CompilerParams(collective_id=0),
    )(x2, Wq, K_loc, V_loc, Wo)


# device time: 12463 ns/iter; 1.1387x vs baseline; 1.0241x over previous
import jax
import jax.numpy as jnp
from jax import lax
from jax.experimental import pallas as pl
from jax.experimental.pallas import tpu as pltpu

N_DEV = 4
B, SQ, SKV, DH, D_MODEL = 2, 128, 128, 64, 512
H_LOC = 4
NC = 8
CR = B * SQ // NC
CPB = NC // B


def kernel(x, Wq, K_ext, V_ext, Wo):
    my = lax.axis_index("i")
    K_loc = lax.dynamic_slice_in_dim(K_ext, my * H_LOC, H_LOC, axis=2)
    V_loc = lax.dynamic_slice_in_dim(V_ext, my * H_LOC, H_LOC, axis=2)
    K_loc = jnp.transpose(K_loc, (0, 2, 1, 3))
    V_loc = jnp.transpose(V_loc, (0, 2, 1, 3))
    x2 = x.reshape(B * SQ, D_MODEL)

    def body(x_ref, wq_ref, k_ref, v_ref, wo_ref, out_ref,
             comm_ref, fsum_ref, send_sems, recv_sems, out_sems):
        my_pos = lax.axis_index("i")
        xp = my_pos ^ 1
        yp = 3 - my_pos

        barrier_sem = pltpu.get_barrier_semaphore()
        for nbr in (xp, yp):
            pl.semaphore_signal(
                barrier_sem, inc=1,
                device_id=(nbr,), device_id_type=pl.DeviceIdType.MESH,
            )

        def exchange(src_slot, dst_slot, sem_idx, partner):
            return pltpu.make_async_remote_copy(
                src_ref=comm_ref.at[src_slot],
                dst_ref=comm_ref.at[dst_slot],
                send_sem=send_sems.at[sem_idx],
                recv_sem=recv_sems.at[sem_idx],
                device_id=(partner,),
                device_id_type=pl.DeviceIdType.MESH,
            )

        q2 = jnp.dot(x_ref[:], wq_ref[:],
                     preferred_element_type=jnp.float32) * 0.125

        stage1 = {}
        stage2 = {}
        for b in range(B):
            ss = []
            for h in range(H_LOC):
                qbh = q2[b * SQ:(b + 1) * SQ, h * DH:(h + 1) * DH]
                ss.append(lax.dot_general(
                    qbh, k_ref[b, h], (((1,), (1,)), ((), ())),
                    preferred_element_type=jnp.float32,
                ))
            s_all = jnp.concatenate(ss, axis=0)
            m = jnp.max(s_all, axis=-1, keepdims=True)
            e = jnp.exp(s_all - m)
            w_all = e / jnp.sum(e, axis=-1, keepdims=True)
            ctxs = [
                jnp.dot(w_all[h * SQ:(h + 1) * SQ, :], v_ref[b, h],
                        preferred_element_type=jnp.float32)
                for h in range(H_LOC)
            ]
            ctx_b = jnp.concatenate(ctxs, axis=-1)
            p_b = jnp.dot(ctx_b, wo_ref[:],
                          preferred_element_type=jnp.float32)
            if b == 0:
                pl.semaphore_wait(barrier_sem, 2)
            for j in range(CPB):
                ci = CPB * b + j
                comm_ref[4 * ci] = p_b[j * CR:(j + 1) * CR, :].astype(
                    jnp.bfloat16)
                r = exchange(4 * ci, 4 * ci + 1, 2 * ci,
                             xp if ci % 2 == 0 else yp)
                r.start()
                stage1[ci] = r

        for ci in range(NC):
            stage1[ci].wait_recv()
            comm_ref[4 * ci + 2] = (
                comm_ref[4 * ci].astype(jnp.float32)
                + comm_ref[4 * ci + 1].astype(jnp.float32)
            ).astype(jnp.bfloat16)
            r = exchange(4 * ci + 2, 4 * ci + 3, 2 * ci + 1,
                         yp if ci % 2 == 0 else xp)
            r.start()
            stage2[ci] = r

        out_copies = []
        for ci in range(NC):
            stage2[ci].wait_recv()
            b, j = divmod(ci, CPB)
            fsum_ref[ci] = (
                comm_ref[4 * ci + 2].astype(jnp.float32)
                + comm_ref[4 * ci + 3].astype(jnp.float32)
            )
            c = pltpu.make_async_copy(
                fsum_ref.at[ci],
                out_ref.at[b, pl.ds(j * CR, CR), :],
                out_sems.at[ci],
            )
            c.start()
            out_copies.append(c)

        for c in out_copies:
            c.wait()
        for r in list(stage1.values()) + list(stage2.values()):
            r.wait_send()

    return pl.pallas_call(
        body,
        out_shape=jax.ShapeDtypeStruct((B, SQ, D_MODEL), jnp.float32),
        in_specs=[pl.BlockSpec(memory_space=pltpu.VMEM)] * 5,
        out_specs=pl.BlockSpec(memory_space=pltpu.MemorySpace.HBM),
        scratch_shapes=[
            pltpu.VMEM((4 * NC, CR, D_MODEL), jnp.bfloat16),
            pltpu.VMEM((NC, CR, D_MODEL), jnp.float32),
            pltpu.SemaphoreType.DMA((2 * NC,)),
            pltpu.SemaphoreType.DMA((2 * NC,)),
            pltpu.SemaphoreType.DMA((NC,)),
        ],
        compiler_params=pltpu.CompilerParams(collective_id=0),
    )(x2, Wq, K_loc, V_loc, Wo)
